# baseline (speedup 1.0000x reference)
import functools

import jax
import jax.numpy as jnp
from jax import lax
from jax.experimental import pallas as pl
from jax.experimental.pallas import tpu as pltpu
from jax.experimental.pallas import tpu_sc as plsc


def _sc_gather(table, idx):
    n_idx = idx.shape[0]
    h = table.shape[1]
    info = plsc.get_sparse_core_info()
    nw = info.num_cores * info.num_subcores
    b_per_w = n_idx // nw
    nc = info.num_cores

    def body(table_hbm, idx_hbm, out_hbm, idx_v, rows_v, sem):
        wid = lax.axis_index("s") * nc + lax.axis_index("c")
        base = wid * b_per_w
        pltpu.sync_copy(idx_hbm.at[pl.ds(base, b_per_w)], idx_v)
        pltpu.async_copy(table_hbm.at[idx_v], rows_v, sem).wait()
        pltpu.sync_copy(rows_v, out_hbm.at[pl.ds(base, b_per_w)])

    gather = pl.kernel(
        body,
        out_type=jax.ShapeDtypeStruct((n_idx, h), table.dtype),
        mesh=plsc.VectorSubcoreMesh(core_axis_name="c", subcore_axis_name="s"),
        scratch_types=[
            pltpu.VMEM((b_per_w,), jnp.int32),
            pltpu.VMEM((b_per_w, h), table.dtype),
            pltpu.SemaphoreType.DMA,
        ],
        compiler_params=pltpu.CompilerParams(use_tc_tiling_on_sc=False),
    )
    return gather(table, idx)


def _mlp_body(e_ref, w1_ref, b1_ref, w2_ref, b2_ref, out_ref, h_ref):
    @pl.when(pl.program_id(0) == 0)
    def _():
        h = lax.dot_general(
            e_ref[...], w1_ref[...], (((1,), (1,)), ((), ())),
            preferred_element_type=jnp.float32)
        h_ref[...] = jnp.maximum(h + b1_ref[...], 0.0)

    out_ref[...] = lax.dot_general(
        h_ref[...], w2_ref[...], (((1,), (1,)), ((), ())),
        preferred_element_type=jnp.float32) + b2_ref[...]


def kernel(x, embed_table, w1, b1, w2, b2):
    batch, ids_per_row = x.shape
    p, hidden = embed_table.shape
    e_dim = ids_per_row * hidden

    idx = x.reshape(-1).astype(jnp.int32)
    e = _sc_gather(embed_table, idx).reshape(batch, e_dim)

    bp = 2048
    np_blocks = (p + bp - 1) // bp

    out = pl.pallas_call(
        _mlp_body,
        grid=(np_blocks,),
        in_specs=[
            pl.BlockSpec((batch, e_dim), lambda i: (0, 0)),
            pl.BlockSpec((hidden, e_dim), lambda i: (0, 0)),
            pl.BlockSpec((1, hidden), lambda i: (0, 0)),
            pl.BlockSpec((bp, hidden), lambda i: (i, 0)),
            pl.BlockSpec((1, bp), lambda i: (0, i)),
        ],
        out_specs=pl.BlockSpec((batch, bp), lambda i: (0, i)),
        out_shape=jax.ShapeDtypeStruct((batch, p), jnp.float32),
        scratch_shapes=[pltpu.VMEM((batch, hidden), jnp.float32)],
    )(e, w1, b1[None, :], w2, b2[None, :])
    return out


# speedup vs baseline: 2.9675x; 2.9675x over previous
import jax
import jax.numpy as jnp
from jax import lax
from jax.experimental import pallas as pl
from jax.experimental.pallas import tpu as pltpu
from jax.experimental.pallas import tpu_sc as plsc


def _sc_gather(table, idx):
    n_idx = idx.shape[0]
    h = table.shape[1]
    info = plsc.get_sparse_core_info()
    nw = info.num_cores * info.num_subcores
    b_per_w = n_idx // nw
    nc = info.num_cores

    def body(table_hbm, idx_hbm, out_hbm, idx_v, rows_v, sem):
        wid = lax.axis_index("s") * nc + lax.axis_index("c")
        base = wid * b_per_w
        pltpu.sync_copy(idx_hbm.at[pl.ds(base, b_per_w)], idx_v)
        pltpu.async_copy(table_hbm.at[idx_v], rows_v, sem).wait()
        pltpu.sync_copy(rows_v, out_hbm.at[pl.ds(base, b_per_w)])

    gather = pl.kernel(
        body,
        out_type=jax.ShapeDtypeStruct((n_idx, h), table.dtype),
        mesh=plsc.VectorSubcoreMesh(core_axis_name="c", subcore_axis_name="s"),
        scratch_types=[
            pltpu.VMEM((b_per_w,), jnp.int32),
            pltpu.VMEM((b_per_w, h), table.dtype),
            pltpu.SemaphoreType.DMA,
        ],
        compiler_params=pltpu.CompilerParams(use_tc_tiling_on_sc=False),
    )
    return gather(table, idx)


def _mlp_body(e_ref, w1_ref, b1_ref, w2t_ref, b2_ref, out_ref, h_ref):
    @pl.when(pl.program_id(0) == 0)
    def _():
        h = lax.dot_general(
            e_ref[...], w1_ref[...], (((1,), (1,)), ((), ())),
            preferred_element_type=jnp.float32)
        h_ref[...] = jnp.maximum(h + b1_ref[...], 0.0)

    batch = h_ref.shape[0]
    bias = lax.dot_general(
        b2_ref[...], jnp.ones((1, batch), jnp.float32), (((0,), (0,)), ((), ())),
        preferred_element_type=jnp.float32)
    out_ref[...] = lax.dot_general(
        w2t_ref[...], h_ref[...], (((0,), (1,)), ((), ())),
        preferred_element_type=jnp.float32) + bias


def kernel(x, embed_table, w1, b1, w2, b2):
    batch, ids_per_row = x.shape
    p, hidden = embed_table.shape
    e_dim = ids_per_row * hidden

    idx = x.reshape(-1).astype(jnp.int32)
    e = _sc_gather(embed_table, idx).reshape(batch, e_dim)

    bp = 2048
    np_blocks = (p + bp - 1) // bp

    out_t = pl.pallas_call(
        _mlp_body,
        grid=(np_blocks,),
        in_specs=[
            pl.BlockSpec((batch, e_dim), lambda i: (0, 0)),
            pl.BlockSpec((hidden, e_dim), lambda i: (0, 0)),
            pl.BlockSpec((1, hidden), lambda i: (0, 0)),
            pl.BlockSpec((hidden, bp), lambda i: (0, i)),
            pl.BlockSpec((1, bp), lambda i: (0, i)),
        ],
        out_specs=pl.BlockSpec((bp, batch), lambda i: (i, 0)),
        out_shape=jax.ShapeDtypeStruct((p, batch), jnp.float32),
        scratch_shapes=[pltpu.VMEM((batch, hidden), jnp.float32)],
    )(e, w1, b1[None, :], w2.T, b2[None, :])
    return out_t.T
